# Initial kernel scaffold; baseline (speedup 1.0000x reference)
import jax
import jax.numpy as jnp
from jax.experimental import pallas as pl


def kernel(sent_a, W):
    raise NotImplementedError("write your pallas kernel here")


# speedup vs baseline: 7.7923x; 7.7923x over previous
import functools

import jax
import jax.numpy as jnp
from jax import lax
from jax.experimental import pallas as pl
from jax.experimental.pallas import tpu as pltpu
from jax.experimental.pallas import tpu_sc as plsc

VOCAB = 100000
EMB = 128
B = 4096
L = 200

_info = plsc.get_sparse_core_info()
NC, NS, LANES = _info.num_cores, _info.num_subcores, _info.num_lanes
NW = NC * NS
BAGS_PER_W = B // NW
C0 = 128
C1 = L - C0
NREG = EMB // LANES


def _ebag_body(sent_hbm, w_hbm, out_hbm, idx_v, buf_v, out_v, sem):
    wid = lax.axis_index("s") * NC + lax.axis_index("c")
    base = wid * BAGS_PER_W

    pltpu.sync_copy(sent_hbm.at[pl.ds(base, BAGS_PER_W)], idx_v)

    def bag_body(i, carry):
        c0 = pltpu.make_async_copy(
            w_hbm.at[idx_v.at[i, pl.ds(0, C0)]], buf_v.at[pl.ds(0, C0)], sem)
        c1 = pltpu.make_async_copy(
            w_hbm.at[idx_v.at[i, pl.ds(C0, C1)]], buf_v.at[pl.ds(C0, C1)], sem)
        c0.start()
        c1.start()
        c0.wait()
        c1.wait()

        def row_body(j, acc):
            return tuple(
                a + buf_v[j, pl.ds(k * LANES, LANES)]
                for k, a in enumerate(acc))

        acc = lax.fori_loop(
            0, L, row_body,
            tuple(jnp.zeros((LANES,), jnp.float32) for _ in range(NREG)))
        for k in range(NREG):
            out_v[i, pl.ds(k * LANES, LANES)] = acc[k]
        return carry

    lax.fori_loop(0, BAGS_PER_W, bag_body, 0)
    pltpu.sync_copy(out_v, out_hbm.at[pl.ds(base, BAGS_PER_W)])


def kernel(sent_a, W):
    sent_a = sent_a.astype(jnp.int32)
    mesh = plsc.VectorSubcoreMesh(core_axis_name="c", subcore_axis_name="s")
    run = functools.partial(
        pl.kernel,
        mesh=mesh,
        out_type=jax.ShapeDtypeStruct((B, EMB), jnp.float32),
        scratch_types=[
            pltpu.VMEM((BAGS_PER_W, L), jnp.int32),
            pltpu.VMEM((L, EMB), jnp.float32),
            pltpu.VMEM((BAGS_PER_W, EMB), jnp.float32),
            pltpu.SemaphoreType.DMA,
        ],
    )(_ebag_body)
    return run(sent_a, W)


# speedup vs baseline: 13.8903x; 1.7826x over previous
import functools

import jax
import jax.numpy as jnp
from jax import lax
from jax.experimental import pallas as pl
from jax.experimental.pallas import tpu as pltpu
from jax.experimental.pallas import tpu_sc as plsc

VOCAB = 100000
EMB = 128
B = 4096
L = 200

_info = plsc.get_sparse_core_info()
NC, NS, LANES = _info.num_cores, _info.num_subcores, _info.num_lanes
NW = NC * NS
BAGS_PER_W = B // NW
C0 = 128
C1 = L - C0
NREG = EMB // LANES


def _ebag_body(sent_hbm, w_hbm, out_hbm, idx_v, buf_v, out_v, sems):
    wid = lax.axis_index("s") * NC + lax.axis_index("c")
    base = wid * BAGS_PER_W

    pltpu.sync_copy(sent_hbm.at[pl.ds(base, BAGS_PER_W)], idx_v)

    def gather_copies(i, slot):
        c0 = pltpu.make_async_copy(
            w_hbm.at[idx_v.at[i, pl.ds(0, C0)]],
            buf_v.at[slot, pl.ds(0, C0)], sems.at[slot])
        c1 = pltpu.make_async_copy(
            w_hbm.at[idx_v.at[i, pl.ds(C0, C1)]],
            buf_v.at[slot, pl.ds(C0, C1)], sems.at[slot])
        return c0, c1

    def start_gather(i, slot):
        c0, c1 = gather_copies(i, slot)
        c0.start()
        c1.start()

    start_gather(0, 0)

    def bag_body(i, carry):
        slot = lax.rem(i, 2)

        @pl.when(i + 1 < BAGS_PER_W)
        def _():
            start_gather(i + 1, 1 - slot)

        c0, c1 = gather_copies(i, slot)
        c0.wait()
        c1.wait()

        def row_body(j, acc):
            return tuple(
                a + buf_v[slot, j, pl.ds(k * LANES, LANES)]
                for k, a in enumerate(acc))

        acc = lax.fori_loop(
            0, L, row_body,
            tuple(jnp.zeros((LANES,), jnp.float32) for _ in range(NREG)))
        for k in range(NREG):
            out_v[i, pl.ds(k * LANES, LANES)] = acc[k]
        return carry

    lax.fori_loop(0, BAGS_PER_W, bag_body, 0)
    pltpu.sync_copy(out_v, out_hbm.at[pl.ds(base, BAGS_PER_W)])


def kernel(sent_a, W):
    sent_a = sent_a.astype(jnp.int32)
    mesh = plsc.VectorSubcoreMesh(core_axis_name="c", subcore_axis_name="s")
    run = functools.partial(
        pl.kernel,
        mesh=mesh,
        out_type=jax.ShapeDtypeStruct((B, EMB), jnp.float32),
        scratch_types=[
            pltpu.VMEM((BAGS_PER_W, L), jnp.int32),
            pltpu.VMEM((2, L, EMB), jnp.float32),
            pltpu.VMEM((BAGS_PER_W, EMB), jnp.float32),
            pltpu.SemaphoreType.DMA((2,)),
        ],
    )(_ebag_body)
    return run(sent_a, W)
